# Initial kernel scaffold; baseline (speedup 1.0000x reference)
import jax
import jax.numpy as jnp
from jax.experimental import pallas as pl


def kernel(x, edge_index, W1, b1, gamma1, beta1, W2, b2, gamma2, beta2, W3, b3):
    raise NotImplementedError("write your pallas kernel here")


# speedup vs baseline: 17.3058x; 17.3058x over previous
import functools

import jax
import jax.numpy as jnp
from jax import lax
from jax.experimental import pallas as pl
from jax.experimental.pallas import tpu as pltpu
from jax.experimental.pallas import tpu_sc as plsc

NC = 2
NS = 16
CH = 125
ZCH = 64


def _sc_mesh():
    return plsc.VectorSubcoreMesh(
        core_axis_name="c", subcore_axis_name="s", num_cores=NC, num_subcores=NS
    )


def _make_deg(np_, d, nch):
    nr = np_ // NS

    @functools.partial(
        pl.kernel,
        mesh=_sc_mesh(),
        out_type=jax.ShapeDtypeStruct((NC, np_, d), jnp.float32),
        scratch_types=[
            pltpu.VMEM_SHARED((np_, d), jnp.float32),
            pltpu.VMEM((nch, CH), jnp.int32),
            pltpu.VMEM((CH, d), jnp.float32),
            pltpu.VMEM((ZCH, d), jnp.float32),
        ],
    )
    def deg_kernel(dst_hbm, ones_hbm, zeros_hbm, out_hbm, acc, didx, ones_v, zeros_v):
        c = lax.axis_index("c")
        s = lax.axis_index("s")
        pltpu.sync_copy(zeros_hbm, zeros_v)
        for k in range(nr // ZCH):
            pltpu.sync_copy(zeros_v, acc.at[pl.ds(s * nr + k * ZCH, ZCH)])
        pltpu.sync_copy(ones_hbm, ones_v)
        pltpu.sync_copy(dst_hbm.at[c, s], didx)
        plsc.subcore_barrier()

        def chunk(j, carry):
            pltpu.sync_copy(ones_v, acc.at[didx.at[j]], add=True)
            return carry

        lax.fori_loop(0, nch, chunk, 0)
        plsc.subcore_barrier()
        pltpu.sync_copy(acc.at[pl.ds(s * nr, nr)], out_hbm.at[c, pl.ds(s * nr, nr)])

    return deg_kernel


def _make_spmm(n, np_, d, nch):
    nr = np_ // NS

    @functools.partial(
        pl.kernel,
        mesh=_sc_mesh(),
        out_type=jax.ShapeDtypeStruct((NC, np_, d), jnp.float32),
        scratch_types=[
            pltpu.VMEM_SHARED((np_, d), jnp.float32),
            pltpu.VMEM((nch, CH), jnp.int32),
            pltpu.VMEM((nch, CH), jnp.int32),
            pltpu.VMEM((CH, d), jnp.float32),
            pltpu.VMEM((ZCH, d), jnp.float32),
        ],
    )
    def spmm_kernel(h_hbm, src_hbm, dst_hbm, zeros_hbm, out_hbm,
                    acc, sidx, didx, rows_v, zeros_v):
        c = lax.axis_index("c")
        s = lax.axis_index("s")
        pltpu.sync_copy(zeros_hbm, zeros_v)
        for k in range(nr // ZCH):
            pltpu.sync_copy(zeros_v, acc.at[pl.ds(s * nr + k * ZCH, ZCH)])
        pltpu.sync_copy(src_hbm.at[c, s], sidx)
        pltpu.sync_copy(dst_hbm.at[c, s], didx)
        plsc.subcore_barrier()

        def chunk(j, carry):
            pltpu.sync_copy(h_hbm.at[sidx.at[j]], rows_v)
            pltpu.sync_copy(rows_v, acc.at[didx.at[j]], add=True)
            return carry

        lax.fori_loop(0, nch, chunk, 0)
        plsc.subcore_barrier()
        pltpu.sync_copy(acc.at[pl.ds(s * nr, nr)], out_hbm.at[c, pl.ds(s * nr, nr)])

    return spmm_kernel


def _tc_pre(dego, x, w1):
    def body(dego_ref, x_ref, w1_ref, h1_ref, dinv_ref):
        n = x_ref.shape[0]
        deg = dego_ref[0, :n, 0:1] + dego_ref[1, :n, 0:1] + 1.0
        dinv = lax.rsqrt(jnp.maximum(deg, 1.0))
        dinv_ref[...] = dinv
        xs = x_ref[...] * dinv
        h1_ref[...] = jnp.dot(xs, w1_ref[...], preferred_element_type=jnp.float32)

    n = x.shape[0]
    return pl.pallas_call(
        body,
        out_shape=(
            jax.ShapeDtypeStruct((n, w1.shape[1]), jnp.float32),
            jax.ShapeDtypeStruct((n, 1), jnp.float32),
        ),
    )(dego, x, w1)


def _tc_mid(p, hp, dinv, b, gamma, beta, wn):
    def body(p_ref, hp_ref, dinv_ref, b_ref, g_ref, be_ref, wn_ref, hn_ref, r_ref):
        n = hp_ref.shape[0]
        dinv = dinv_ref[...]
        agg = (p_ref[0, :n] + p_ref[1, :n] + hp_ref[...]) * dinv + b_ref[...]
        m = jnp.mean(agg, axis=0, keepdims=True)
        v = jnp.mean((agg - m) ** 2, axis=0, keepdims=True)
        z = (agg - m) * lax.rsqrt(v + 1e-5) * g_ref[...] + be_ref[...]
        r = jnp.maximum(z, 0.0)
        r_ref[...] = r
        hn_ref[...] = jnp.dot(r * dinv, wn_ref[...], preferred_element_type=jnp.float32)

    n, d = hp.shape
    return pl.pallas_call(
        body,
        out_shape=(
            jax.ShapeDtypeStruct((n, wn.shape[1]), jnp.float32),
            jax.ShapeDtypeStruct((n, d), jnp.float32),
        ),
    )(p, hp, dinv, b.reshape(1, -1), gamma.reshape(1, -1), beta.reshape(1, -1), wn)


def _tc_fin(p, hp, dinv, b):
    def body(p_ref, hp_ref, dinv_ref, b_ref, out_ref):
        n = hp_ref.shape[0]
        out_ref[...] = (
            p_ref[0, :n] + p_ref[1, :n] + hp_ref[...]
        ) * dinv_ref[...] + b_ref[...]

    n, d = hp.shape
    return pl.pallas_call(
        body,
        out_shape=jax.ShapeDtypeStruct((n, d), jnp.float32),
    )(p, hp, dinv, b.reshape(1, -1))


def kernel(x, edge_index, W1, b1, gamma1, beta1, W2, b2, gamma2, beta2, W3, b3):
    n, d = x.shape
    e = edge_index.shape[1]
    per_tile = e // (NC * NS)
    nch = per_tile // CH
    np_ = ((n + NS * ZCH - 1) // (NS * ZCH)) * (NS * ZCH)
    assert per_tile % CH == 0 and e == per_tile * NC * NS

    src_r = edge_index[0].reshape(NC, NS, nch, CH)
    dst_r = edge_index[1].reshape(NC, NS, nch, CH)
    onerows = jnp.ones((CH, d), jnp.float32)
    zrows = jnp.zeros((ZCH, d), jnp.float32)

    deg_k = _make_deg(np_, d, nch)
    spmm_k = _make_spmm(n, np_, d, nch)

    dego = deg_k(dst_r, onerows, zrows)
    h1, dinv = _tc_pre(dego, x, W1)

    p1 = spmm_k(h1, src_r, dst_r, zrows)
    h2, _ = _tc_mid(p1, h1, dinv, b1, gamma1, beta1, W2)

    p2 = spmm_k(h2, src_r, dst_r, zrows)
    h3, emb = _tc_mid(p2, h2, dinv, b2, gamma2, beta2, W3)

    p3 = spmm_k(h3, src_r, dst_r, zrows)
    hc = _tc_fin(p3, h3, dinv, b3)
    return (emb, hc)
